# baseline (speedup 1.0000x reference)
import functools

import jax
import jax.numpy as jnp
from jax import lax
from jax.experimental import pallas as pl
from jax.experimental.pallas import tpu as pltpu
from jax.experimental.pallas import tpu_sc as plsc

NC = 2
NS = 16
L = 16
NW = NC * NS

B = 4096
P = 20
N = 50
S = P + N
SP = 72
SPD = 80
D = 128
NT = 4
NTYPES_LEN = 64
EPS = 1e-15

BPW = B // NW
SLOTS = BPW * SP
KD = D // L
NG = SPD // L
NB = 4
BPS = 1
NSTR = BPW // BPS
RBUF = BPS * SP + (SPD - SP)
FLAT_HEAD = 2 * NB

LN2 = 0.6931471805599453
C0 = 9.99828090e-05
C1 = 1.43730442e+00
C2 = -6.72940494e-01
C3 = 3.15473611e-01
C4 = -8.00124786e-02


def _tree_sum(vs):
    while len(vs) > 1:
        vs = [vs[i] + vs[i + 1] for i in range(0, len(vs) - 1, 2)] + (
            [vs[-1]] if len(vs) % 2 else [])
    return vs[0]


def _sc_body(samples_hbm, snode_hbm, types_hbm, nsg_hbm, w_hbm,
             semb_hbm, eemb_hbm,
             loss_hbm,
             samp_v, flat_v, snode_v, types_v, t_v, nsg_v, w_v, srows_v,
             rows0, rows1, rows2, rows3, loss_v, sem0, sem1, sem2, sem3):
    wid = lax.axis_index("s") * NC + lax.axis_index("c")
    base_b = wid * BPW

    pltpu.sync_copy(samples_hbm.at[pl.ds(wid * SLOTS, SLOTS)], samp_v)
    pltpu.sync_copy(snode_hbm.at[pl.ds(base_b, BPW)], snode_v)
    pltpu.sync_copy(types_hbm, types_v)
    pltpu.sync_copy(nsg_hbm, nsg_v)
    pltpu.sync_copy(w_hbm, w_v)

    pltpu.async_copy(semb_hbm.at[snode_v], srows_v, sem0).wait()

    for g in range(BPW // L):
        sn = snode_v[pl.ds(g * L, L)]
        t_v[pl.ds(g * L, L)] = plsc.load_gather(types_v, [sn])

    iota = lax.iota(jnp.int32, L)

    def flat_body(i, c):
        basei = i * L
        lanes = basei + iota
        bloc = lax.div(lanes, SP)
        tt = plsc.load_gather(t_v, [bloc])
        sv = samp_v[pl.ds(basei, L)]
        flat_v[pl.ds(basei, L)] = sv * NT + tt
        return c

    lax.fori_loop(0, FLAT_HEAD * SP // L, flat_body, 0)

    def fire(s, buf, sem):
        pltpu.async_copy(eemb_hbm.at[flat_v.at[pl.ds(s * BPS * SP, BPS * SP)]],
                         buf.at[pl.ds(0, BPS * SP)], sem)

    def drain(s, buf, sem):
        pltpu.make_async_copy(
            eemb_hbm.at[flat_v.at[pl.ds(s * BPS * SP, BPS * SP)]],
            buf.at[pl.ds(0, BPS * SP)], sem).wait()

    masks = [(iota & k) != 0 for k in (1, 2, 4, 8)]
    perms = [iota ^ k for k in (1, 2, 4, 8)]
    m15 = iota == (L - 1)

    _dnums = lax.GatherDimensionNumbers(
        offset_dims=(), collapsed_slice_dims=(0,), start_index_map=(0,))

    def _shuf(v, r):
        return lax.gather(v, perms[r][:, None], _dnums, slice_sizes=(1,),
                          mode=lax.GatherScatterMode.PROMISE_IN_BOUNDS)

    def compute(b, buf, rbase):
        svecs = [srows_v[b, pl.ds(k * L, L)] for k in range(KD)]

        def group_body(g, carry):
            row0 = g * L
            accs = []
            for jj in range(L):
                accs.append(_tree_sum(
                    [buf[rbase + row0 + jj, pl.ds(k * L, L)] * svecs[k]
                     for k in range(KD)]))
            for r in range(4):
                accs = [jnp.where(masks[r], accs[2 * m + 1], accs[2 * m])
                        + _shuf(jnp.where(masks[r], accs[2 * m],
                                          accs[2 * m + 1]), r)
                        for m in range(len(accs) // 2)]
            dvec = accs[0]
            nsg = nsg_v[pl.ds(row0, L)]
            wg = w_v[pl.ds(row0, L)]
            dz = jnp.where(wg != 0.0, dvec, 0.0)
            p = 1.0 / (1.0 + jnp.exp(dz * nsg))
            a = p + EPS
            bits = lax.bitcast_convert_type(a, jnp.int32)
            ef = (jnp.right_shift(bits, 23) - 127).astype(jnp.float32)
            m = lax.bitcast_convert_type(
                jnp.bitwise_or(jnp.bitwise_and(bits, 0x007FFFFF),
                               0x3F800000), jnp.float32)
            z = m - 1.0
            l2m = C0 + z * (C1 + z * (C2 + z * (C3 + z * C4)))
            lg = (ef + l2m) * LN2
            return carry + wg * lg

        wsum = lax.fori_loop(0, NG, group_body, jnp.zeros((L,), jnp.float32))
        c = plsc.cumsum(wsum)
        plsc.store_scatter(loss_v, [jnp.broadcast_to(b, (L,))], -c, mask=m15)

    rings = (rows0, rows1, rows2, rows3)
    sems = (sem0, sem1, sem2, sem3)
    for q in range(NB):
        fire(q, rings[q], sems[q])

    lax.fori_loop(FLAT_HEAD * SP // L, SLOTS // L, flat_body, 0)

    def ring_body(i, c):
        s0 = NB * i
        for q in range(NB):
            s = s0 + q
            drain(s, rings[q], sems[q])
            for u in range(BPS):
                compute(s * BPS + u, rings[q], u * SP)

            @pl.when(s + NB < NSTR)
            def _():
                fire(s + NB, rings[q], sems[q])
        return c

    lax.fori_loop(0, NSTR // NB, ring_body, 0)

    pltpu.sync_copy(loss_v, loss_hbm.at[pl.ds(base_b, BPW)])


@functools.cache
def _sc_loss_fn():
  return functools.partial(
    pl.kernel,
    out_type=jax.ShapeDtypeStruct((B,), jnp.float32),
    mesh=plsc.VectorSubcoreMesh(core_axis_name="c", subcore_axis_name="s",
                                num_cores=NC, num_subcores=NS),
    scratch_types=[
        pltpu.VMEM((SLOTS,), jnp.int32),
        pltpu.VMEM((SLOTS,), jnp.int32),
        pltpu.VMEM((BPW,), jnp.int32),
        pltpu.VMEM((NTYPES_LEN,), jnp.int32),
        pltpu.VMEM((BPW,), jnp.int32),
        pltpu.VMEM((SPD,), jnp.float32),
        pltpu.VMEM((SPD,), jnp.float32),
        pltpu.VMEM((BPW, D), jnp.float32),
        pltpu.VMEM((RBUF, D), jnp.float32),
        pltpu.VMEM((RBUF, D), jnp.float32),
        pltpu.VMEM((RBUF, D), jnp.float32),
        pltpu.VMEM((RBUF, D), jnp.float32),
        pltpu.VMEM((BPW,), jnp.float32),
        pltpu.SemaphoreType.DMA,
        pltpu.SemaphoreType.DMA,
        pltpu.SemaphoreType.DMA,
        pltpu.SemaphoreType.DMA,
    ],
    compiler_params=pltpu.CompilerParams(needs_layout_passes=False),
  )(_sc_body)


def kernel(start_node, pos_samples, neg_samples, start_embeds, end_embeds,
           node_types):
    pad = jnp.zeros((B, SP - S), jnp.int32)
    samples = jnp.concatenate([pos_samples, neg_samples, pad], axis=1)
    samples_flat = samples.reshape(-1)
    snode = start_node.reshape(-1)
    eemb_flat = end_embeds.reshape(-1, D)
    j = jnp.arange(SPD)
    nsg = jnp.where(j < P, -1.0, 1.0).astype(jnp.float32)
    w = jnp.where(j < P, 1.0 / P,
                  jnp.where(j < S, 1.0 / N, 0.0)).astype(jnp.float32)
    return _sc_loss_fn()(samples_flat, snode, node_types, nsg, w,
                         start_embeds, eemb_flat)


# speedup vs baseline: 1.2607x; 1.2607x over previous
import functools

import jax
import jax.numpy as jnp
from jax import lax
from jax.experimental import pallas as pl
from jax.experimental.pallas import tpu as pltpu
from jax.experimental.pallas import tpu_sc as plsc

NC = 2
NS = 16
L = 16
NW = NC * NS

B = 4096
P = 20
N = 50
S = P + N
SP = 72
SPD = 80
D = 128
NT = 4
NTYPES_LEN = 64
EPS = 1e-15

BPW = B // NW
SLOTS = BPW * SP
KD = D // L
NG = SPD // L
NB = 4
BPS = 1
NSTR = BPW // BPS
RBUF = BPS * SP + (SPD - SP)
FLAT_HEAD = 2 * NB

LN2 = 0.6931471805599453
C0 = 9.99828090e-05
C1 = 1.43730442e+00
C2 = -6.72940494e-01
C3 = 3.15473611e-01
C4 = -8.00124786e-02


def _tree_sum(vs):
    while len(vs) > 1:
        vs = [vs[i] + vs[i + 1] for i in range(0, len(vs) - 1, 2)] + (
            [vs[-1]] if len(vs) % 2 else [])
    return vs[0]


def _sc_body(samples_hbm, snode_hbm, types_hbm, nsg_hbm, w_hbm,
             semb_hbm, eemb_hbm,
             loss_hbm,
             samp_v, flat_v, snode_v, types_v, t_v, nsg_v, w_v, srows_v,
             rows0, rows1, rows2, rows3, dotsA, dotsB, loss_v,
             sem0, sem1, sem2, sem3):
    wid = lax.axis_index("s") * NC + lax.axis_index("c")
    base_b = wid * BPW

    pltpu.sync_copy(samples_hbm.at[pl.ds(wid * SLOTS, SLOTS)], samp_v)
    pltpu.sync_copy(snode_hbm.at[pl.ds(base_b, BPW)], snode_v)
    pltpu.sync_copy(types_hbm, types_v)
    pltpu.sync_copy(nsg_hbm, nsg_v)
    pltpu.sync_copy(w_hbm, w_v)

    pltpu.async_copy(semb_hbm.at[snode_v], srows_v, sem0).wait()

    for g in range(BPW // L):
        sn = snode_v[pl.ds(g * L, L)]
        t_v[pl.ds(g * L, L)] = plsc.load_gather(types_v, [sn])

    iota = lax.iota(jnp.int32, L)

    def flat_body(i, c):
        basei = i * L
        lanes = basei + iota
        bloc = lax.div(lanes, SP)
        tt = plsc.load_gather(t_v, [bloc])
        sv = samp_v[pl.ds(basei, L)]
        flat_v[pl.ds(basei, L)] = sv * NT + tt
        return c

    lax.fori_loop(0, FLAT_HEAD * SP // L, flat_body, 0)

    def fire(s, buf, sem):
        pltpu.async_copy(eemb_hbm.at[flat_v.at[pl.ds(s * BPS * SP, BPS * SP)]],
                         buf.at[pl.ds(0, BPS * SP)], sem)

    def drain(s, buf, sem):
        pltpu.make_async_copy(
            eemb_hbm.at[flat_v.at[pl.ds(s * BPS * SP, BPS * SP)]],
            buf.at[pl.ds(0, BPS * SP)], sem).wait()

    masks = [(iota & k) != 0 for k in (1, 2, 4, 8)]
    perms = [iota ^ k for k in (1, 2, 4, 8)]
    m15 = iota == (L - 1)

    _dnums = lax.GatherDimensionNumbers(
        offset_dims=(), collapsed_slice_dims=(0,), start_index_map=(0,))

    def _shuf(v, r):
        return lax.gather(v, perms[r][:, None], _dnums, slice_sizes=(1,),
                          mode=lax.GatherScatterMode.PROMISE_IN_BOUNDS)

    def dots_pass(b, buf, rbase, dbuf):
        svecs = [srows_v[b, pl.ds(k * L, L)] for k in range(KD)]

        def group_body(g, c):
            row0 = g * L
            accs = []
            for jj in range(L):
                accs.append(_tree_sum(
                    [buf[rbase + row0 + jj, pl.ds(k * L, L)] * svecs[k]
                     for k in range(KD)]))
            for r in range(4):
                accs = [jnp.where(masks[r], accs[2 * m + 1], accs[2 * m])
                        + _shuf(jnp.where(masks[r], accs[2 * m],
                                          accs[2 * m + 1]), r)
                        for m in range(len(accs) // 2)]
            dbuf[pl.ds(row0, L)] = accs[0]
            return c

        lax.fori_loop(0, NG, group_body, 0)

    def loss_pass(b, dbuf):
        def group_body(g, carry):
            row0 = g * L
            dvec = dbuf[pl.ds(row0, L)]
            nsg = nsg_v[pl.ds(row0, L)]
            wg = w_v[pl.ds(row0, L)]
            dz = jnp.where(wg != 0.0, dvec, 0.0)
            p = 1.0 / (1.0 + jnp.exp(dz * nsg))
            a = p + EPS
            bits = lax.bitcast_convert_type(a, jnp.int32)
            ef = (jnp.right_shift(bits, 23) - 127).astype(jnp.float32)
            m = lax.bitcast_convert_type(
                jnp.bitwise_or(jnp.bitwise_and(bits, 0x007FFFFF),
                               0x3F800000), jnp.float32)
            z = m - 1.0
            l2m = C0 + z * (C1 + z * (C2 + z * (C3 + z * C4)))
            lg = (ef + l2m) * LN2
            return carry + wg * lg

        wsum = lax.fori_loop(0, NG, group_body, jnp.zeros((L,), jnp.float32))
        c = plsc.cumsum(wsum)
        valid = m15 & (jnp.broadcast_to(b, (L,)) >= 0)
        plsc.store_scatter(loss_v, [jnp.broadcast_to(b, (L,))], -c,
                           mask=valid)

    rings = (rows0, rows1, rows2, rows3)
    sems = (sem0, sem1, sem2, sem3)
    dbufs = (dotsA, dotsB)
    for q in range(NB):
        fire(q, rings[q], sems[q])

    lax.fori_loop(FLAT_HEAD * SP // L, SLOTS // L, flat_body, 0)

    def ring_body(i, c):
        s0 = NB * i
        for q in range(NB):
            s = s0 + q
            loss_pass(s - 1, dbufs[(q + 1) % 2])
            drain(s, rings[q], sems[q])
            dots_pass(s, rings[q], 0, dbufs[q % 2])

            @pl.when(s + NB < NSTR)
            def _():
                fire(s + NB, rings[q], sems[q])
        return c

    lax.fori_loop(0, NSTR // NB, ring_body, 0)
    loss_pass(NSTR - 1, dbufs[(NSTR - 1) % 2])

    pltpu.sync_copy(loss_v, loss_hbm.at[pl.ds(base_b, BPW)])


@functools.cache
def _sc_loss_fn():
  return functools.partial(
    pl.kernel,
    out_type=jax.ShapeDtypeStruct((B,), jnp.float32),
    mesh=plsc.VectorSubcoreMesh(core_axis_name="c", subcore_axis_name="s",
                                num_cores=NC, num_subcores=NS),
    scratch_types=[
        pltpu.VMEM((SLOTS,), jnp.int32),
        pltpu.VMEM((SLOTS,), jnp.int32),
        pltpu.VMEM((BPW,), jnp.int32),
        pltpu.VMEM((NTYPES_LEN,), jnp.int32),
        pltpu.VMEM((BPW,), jnp.int32),
        pltpu.VMEM((SPD,), jnp.float32),
        pltpu.VMEM((SPD,), jnp.float32),
        pltpu.VMEM((BPW, D), jnp.float32),
        pltpu.VMEM((RBUF, D), jnp.float32),
        pltpu.VMEM((RBUF, D), jnp.float32),
        pltpu.VMEM((RBUF, D), jnp.float32),
        pltpu.VMEM((RBUF, D), jnp.float32),
        pltpu.VMEM((SPD,), jnp.float32),
        pltpu.VMEM((SPD,), jnp.float32),
        pltpu.VMEM((BPW,), jnp.float32),
        pltpu.SemaphoreType.DMA,
        pltpu.SemaphoreType.DMA,
        pltpu.SemaphoreType.DMA,
        pltpu.SemaphoreType.DMA,
    ],
    compiler_params=pltpu.CompilerParams(needs_layout_passes=False),
  )(_sc_body)


def kernel(start_node, pos_samples, neg_samples, start_embeds, end_embeds,
           node_types):
    pad = jnp.zeros((B, SP - S), jnp.int32)
    samples = jnp.concatenate([pos_samples, neg_samples, pad], axis=1)
    samples_flat = samples.reshape(-1)
    snode = start_node.reshape(-1)
    eemb_flat = end_embeds.reshape(-1, D)
    j = jnp.arange(SPD)
    nsg = jnp.where(j < P, -1.0, 1.0).astype(jnp.float32)
    w = jnp.where(j < P, 1.0 / P,
                  jnp.where(j < S, 1.0 / N, 0.0)).astype(jnp.float32)
    return _sc_loss_fn()(samples_flat, snode, node_types, nsg, w,
                         start_embeds, eemb_flat)


# speedup vs baseline: 1.6379x; 1.2992x over previous
import functools

import jax
import jax.numpy as jnp
from jax import lax
from jax.experimental import pallas as pl
from jax.experimental.pallas import tpu as pltpu
from jax.experimental.pallas import tpu_sc as plsc

NC = 2
NS = 16
L = 16
NW = NC * NS

B = 4096
P = 20
N = 50
S = P + N
SP = 72
SPD = 80
D = 128
NT = 4
NTYPES_LEN = 64
EPS = 1e-15

BPW = B // NW
SLOTS = BPW * SP
KD = D // L
NG = SPD // L
NB = 4
BPS = 1
NSTR = BPW // BPS
RBUF = BPS * SP + (SPD - SP)

LN2 = 0.6931471805599453
C0 = 9.99828090e-05
C1 = 1.43730442e+00
C2 = -6.72940494e-01
C3 = 3.15473611e-01
C4 = -8.00124786e-02


def _tree_sum(vs):
    while len(vs) > 1:
        vs = [vs[i] + vs[i + 1] for i in range(0, len(vs) - 1, 2)] + (
            [vs[-1]] if len(vs) % 2 else [])
    return vs[0]


def _sc_body(samples_hbm, snode_hbm, types_hbm, nsg_hbm, w_hbm,
             semb_hbm, eemb_hbm,
             loss_hbm,
             samp_v, flat_v, snode_v, types_v, t_v, nsg_v, w_v, srows_v,
             rows0, rows1, rows2, rows3, dotsA, dotsB, loss_v,
             sem0, sem1, sem2, sem3):
    wid = lax.axis_index("s") * NC + lax.axis_index("c")
    base_b = wid * BPW

    pltpu.sync_copy(samples_hbm.at[pl.ds(wid * SLOTS, SLOTS)],
                    samp_v.at[pl.ds(0, SLOTS)])
    pltpu.sync_copy(snode_hbm.at[pl.ds(base_b, BPW)], snode_v)
    pltpu.sync_copy(types_hbm, types_v)
    pltpu.sync_copy(nsg_hbm, nsg_v)
    pltpu.sync_copy(w_hbm, w_v)

    pltpu.async_copy(semb_hbm.at[snode_v], srows_v, sem0).wait()

    for g in range(BPW // L):
        sn = snode_v[pl.ds(g * L, L)]
        t_v[pl.ds(g * L, L)] = plsc.load_gather(types_v, [sn])

    iota = lax.iota(jnp.int32, L)

    def flat_for_stream(s):
        for gg in range(NG):
            basei = s * SP + gg * L
            lanes = basei + iota
            bloc = lax.div(lanes, SP)
            tt = plsc.load_gather(t_v, [bloc])
            sv = samp_v[pl.ds(basei, L)]
            flat_v[pl.ds(basei, L)] = sv * NT + tt

    def fire(s, buf, sem):
        pltpu.async_copy(eemb_hbm.at[flat_v.at[pl.ds(s * BPS * SP, BPS * SP)]],
                         buf.at[pl.ds(0, BPS * SP)], sem)

    def drain(s, buf, sem):
        pltpu.make_async_copy(
            eemb_hbm.at[flat_v.at[pl.ds(s * BPS * SP, BPS * SP)]],
            buf.at[pl.ds(0, BPS * SP)], sem).wait()

    masks = [(iota & k) != 0 for k in (1, 2, 4, 8)]
    perms = [iota ^ k for k in (1, 2, 4, 8)]
    m15 = iota == (L - 1)

    _dnums = lax.GatherDimensionNumbers(
        offset_dims=(), collapsed_slice_dims=(0,), start_index_map=(0,))

    def _shuf(v, r):
        return lax.gather(v, perms[r][:, None], _dnums, slice_sizes=(1,),
                          mode=lax.GatherScatterMode.PROMISE_IN_BOUNDS)

    def dots_pass(b, buf, rbase, dbuf):
        svecs = [srows_v[b, pl.ds(k * L, L)] for k in range(KD)]

        def group_body(g, c):
            row0 = g * L
            accs = []
            for jj in range(L):
                accs.append(_tree_sum(
                    [buf[rbase + row0 + jj, pl.ds(k * L, L)] * svecs[k]
                     for k in range(KD)]))
            for r in range(4):
                accs = [jnp.where(masks[r], accs[2 * m + 1], accs[2 * m])
                        + _shuf(jnp.where(masks[r], accs[2 * m],
                                          accs[2 * m + 1]), r)
                        for m in range(len(accs) // 2)]
            dbuf[pl.ds(row0, L)] = accs[0]
            return c

        lax.fori_loop(0, NG, group_body, 0)

    def loss_pass(b, dbuf):
        def group_body(g, carry):
            row0 = g * L
            dvec = dbuf[pl.ds(row0, L)]
            nsg = nsg_v[pl.ds(row0, L)]
            wg = w_v[pl.ds(row0, L)]
            dz = jnp.where(wg != 0.0, dvec, 0.0)
            p = 1.0 / (1.0 + jnp.exp(dz * nsg))
            a = p + EPS
            bits = lax.bitcast_convert_type(a, jnp.int32)
            ef = (jnp.right_shift(bits, 23) - 127).astype(jnp.float32)
            m = lax.bitcast_convert_type(
                jnp.bitwise_or(jnp.bitwise_and(bits, 0x007FFFFF),
                               0x3F800000), jnp.float32)
            z = m - 1.0
            l2m = C0 + z * (C1 + z * (C2 + z * (C3 + z * C4)))
            lg = (ef + l2m) * LN2
            return carry + wg * lg

        wsum = lax.fori_loop(0, NG, group_body, jnp.zeros((L,), jnp.float32))
        c = plsc.cumsum(wsum)
        valid = m15 & (jnp.broadcast_to(b, (L,)) >= 0)
        plsc.store_scatter(loss_v, [jnp.broadcast_to(b, (L,))], -c,
                           mask=valid)

    rings = (rows0, rows1, rows2, rows3)
    sems = (sem0, sem1, sem2, sem3)
    dbufs = (dotsA, dotsB)
    for q in range(NB):
        flat_for_stream(q)
        fire(q, rings[q], sems[q])

    def ring_body(i, c):
        s0 = NB * i
        for q in range(NB):
            s = s0 + q
            loss_pass(s - 1, dbufs[(q + 1) % 2])
            drain(s, rings[q], sems[q])
            dots_pass(s, rings[q], 0, dbufs[q % 2])

            @pl.when(s + NB < NSTR)
            def _():
                flat_for_stream(s + NB)
                fire(s + NB, rings[q], sems[q])
        return c

    lax.fori_loop(0, NSTR // NB, ring_body, 0)
    loss_pass(NSTR - 1, dbufs[(NSTR - 1) % 2])

    pltpu.sync_copy(loss_v, loss_hbm.at[pl.ds(base_b, BPW)])


@functools.cache
def _sc_loss_fn():
  return functools.partial(
    pl.kernel,
    out_type=jax.ShapeDtypeStruct((B,), jnp.float32),
    mesh=plsc.VectorSubcoreMesh(core_axis_name="c", subcore_axis_name="s",
                                num_cores=NC, num_subcores=NS),
    scratch_types=[
        pltpu.VMEM((SLOTS + L,), jnp.int32),
        pltpu.VMEM((SLOTS + L,), jnp.int32),
        pltpu.VMEM((BPW,), jnp.int32),
        pltpu.VMEM((NTYPES_LEN,), jnp.int32),
        pltpu.VMEM((BPW + L,), jnp.int32),
        pltpu.VMEM((SPD,), jnp.float32),
        pltpu.VMEM((SPD,), jnp.float32),
        pltpu.VMEM((BPW, D), jnp.float32),
        pltpu.VMEM((RBUF, D), jnp.float32),
        pltpu.VMEM((RBUF, D), jnp.float32),
        pltpu.VMEM((RBUF, D), jnp.float32),
        pltpu.VMEM((RBUF, D), jnp.float32),
        pltpu.VMEM((SPD,), jnp.float32),
        pltpu.VMEM((SPD,), jnp.float32),
        pltpu.VMEM((BPW,), jnp.float32),
        pltpu.SemaphoreType.DMA,
        pltpu.SemaphoreType.DMA,
        pltpu.SemaphoreType.DMA,
        pltpu.SemaphoreType.DMA,
    ],
    compiler_params=pltpu.CompilerParams(needs_layout_passes=False),
  )(_sc_body)


def kernel(start_node, pos_samples, neg_samples, start_embeds, end_embeds,
           node_types):
    pad = jnp.zeros((B, SP - S), jnp.int32)
    samples = jnp.concatenate([pos_samples, neg_samples, pad], axis=1)
    samples_flat = samples.reshape(-1)
    snode = start_node.reshape(-1)
    eemb_flat = end_embeds.reshape(-1, D)
    j = jnp.arange(SPD)
    nsg = jnp.where(j < P, -1.0, 1.0).astype(jnp.float32)
    w = jnp.where(j < P, 1.0 / P,
                  jnp.where(j < S, 1.0 / N, 0.0)).astype(jnp.float32)
    return _sc_loss_fn()(samples_flat, snode, node_types, nsg, w,
                         start_embeds, eemb_flat)
